# Initial kernel scaffold; baseline (speedup 1.0000x reference)
import jax
import jax.numpy as jnp
from jax.experimental import pallas as pl


def kernel(x):
    raise NotImplementedError("write your pallas kernel here")


# speedup vs baseline: 1.2874x; 1.2874x over previous
import jax
import jax.numpy as jnp
from jax.experimental import pallas as pl
from jax.experimental.pallas import tpu as pltpu

_BLOCK_ROWS = 4096


def _norm_body(x_ref, o_ref):
    x = x_ref[...]
    s = jnp.sum(x, axis=1, keepdims=True)
    o_ref[...] = x * (1.0 / s)


def kernel(x):
    n, b = x.shape
    grid = (n // _BLOCK_ROWS,)
    return pl.pallas_call(
        _norm_body,
        grid=grid,
        in_specs=[pl.BlockSpec((_BLOCK_ROWS, b), lambda i: (i, 0))],
        out_specs=pl.BlockSpec((_BLOCK_ROWS, b), lambda i: (i, 0)),
        out_shape=jax.ShapeDtypeStruct(x.shape, x.dtype),
        compiler_params=pltpu.CompilerParams(
            dimension_semantics=("parallel",),
        ),
    )(x)


# speedup vs baseline: 1.5318x; 1.1898x over previous
import jax
import jax.numpy as jnp
from jax.experimental import pallas as pl
from jax.experimental.pallas import tpu as pltpu

_BLOCK_ROWS = 8192


def _norm_body(x_ref, o_ref):
    x = x_ref[...]
    s = jnp.sum(x, axis=1, keepdims=True)
    o_ref[...] = x * (1.0 / s)


def kernel(x):
    n, b = x.shape
    grid = (n // _BLOCK_ROWS,)
    return pl.pallas_call(
        _norm_body,
        grid=grid,
        in_specs=[pl.BlockSpec((_BLOCK_ROWS, b), lambda i: (i, 0))],
        out_specs=pl.BlockSpec((_BLOCK_ROWS, b), lambda i: (i, 0)),
        out_shape=jax.ShapeDtypeStruct(x.shape, x.dtype),
        compiler_params=pltpu.CompilerParams(
            dimension_semantics=("parallel",),
        ),
    )(x)


# speedup vs baseline: 1.5348x; 1.0020x over previous
import jax
import jax.numpy as jnp
from jax.experimental import pallas as pl
from jax.experimental.pallas import tpu as pltpu

_BLOCK_ROWS = 16384


def _norm_body(x_ref, o_ref):
    x = x_ref[...]
    s = jnp.sum(x, axis=1, keepdims=True)
    o_ref[...] = x * (1.0 / s)


def kernel(x):
    n, b = x.shape
    grid = (n // _BLOCK_ROWS,)
    return pl.pallas_call(
        _norm_body,
        grid=grid,
        in_specs=[pl.BlockSpec((_BLOCK_ROWS, b), lambda i: (i, 0))],
        out_specs=pl.BlockSpec((_BLOCK_ROWS, b), lambda i: (i, 0)),
        out_shape=jax.ShapeDtypeStruct(x.shape, x.dtype),
        compiler_params=pltpu.CompilerParams(
            dimension_semantics=("parallel",),
        ),
    )(x)


# speedup vs baseline: 1.5742x; 1.0257x over previous
import jax
import jax.numpy as jnp
from jax.experimental import pallas as pl
from jax.experimental.pallas import tpu as pltpu

_BLOCK_ROWS = 29960


def _norm_body(x_ref, o_ref):
    x = x_ref[...]
    s = jnp.sum(x, axis=1, keepdims=True)
    o_ref[...] = x * (1.0 / s)


def kernel(x):
    n, b = x.shape
    grid = (pl.cdiv(n, _BLOCK_ROWS),)
    return pl.pallas_call(
        _norm_body,
        grid=grid,
        in_specs=[pl.BlockSpec((_BLOCK_ROWS, b), lambda i: (i, 0))],
        out_specs=pl.BlockSpec((_BLOCK_ROWS, b), lambda i: (i, 0)),
        out_shape=jax.ShapeDtypeStruct(x.shape, x.dtype),
        compiler_params=pltpu.CompilerParams(
            dimension_semantics=("parallel",),
        ),
    )(x)
